# baseline (speedup 1.0000x reference)
import functools

import jax
import jax.numpy as jnp
from jax import lax
from jax.experimental import pallas as pl
from jax.experimental.pallas import tpu as pltpu
from jax.experimental.pallas import tpu_sc as plsc

N = 10000
E = 320000
D_IN = 128
D_H = 256
N_CLASSES = 10
N_GRAPHS = 128

NC, NS = 2, 16
CHUNK = 128
NCHUNK = 160
EPT = NCHUNK * CHUNK
EPAD = NS * EPT
NPAD = 10240
DUMMY = N
ROWS_PER_TILE = NPAD // NS
DH = 128

_mesh = plsc.VectorSubcoreMesh(
    core_axis_name="c", subcore_axis_name="s", num_cores=NC, num_subcores=NS)


def _deg_body(dst_hbm, ones_hbm, zeros_hbm, out_hbm, acc, dstv, onesv):
  c = lax.axis_index("c")
  s = lax.axis_index("s")
  pltpu.sync_copy(dst_hbm.at[s], dstv)
  pltpu.sync_copy(ones_hbm, onesv)
  rows = pl.ds(s * ROWS_PER_TILE, ROWS_PER_TILE)
  pltpu.sync_copy(zeros_hbm, acc.at[rows])
  plsc.subcore_barrier()

  base = c * (NCHUNK // 2)

  def hist_body(j, _):
    pltpu.sync_copy(onesv, acc.at[dstv.at[base + j]], add=True)
    return 0
  lax.fori_loop(0, NCHUNK // 2, hist_body, 0)

  plsc.subcore_barrier()
  pltpu.sync_copy(acc.at[rows], out_hbm.at[c].at[rows])


def _deg(dst3):
  ones = jnp.ones((CHUNK, DH), jnp.float32)
  zeros = jnp.zeros((ROWS_PER_TILE, DH), jnp.float32)
  return pl.kernel(
      _deg_body,
      out_type=jax.ShapeDtypeStruct((NC, NPAD, DH), jnp.float32),
      mesh=_mesh,
      scratch_types=[
          pltpu.VMEM_SHARED((NPAD, DH), jnp.float32),
          pltpu.VMEM((NCHUNK, CHUNK), jnp.int32),
          pltpu.VMEM((CHUNK, DH), jnp.float32),
      ],
  )(dst3, ones, zeros)


SK = 16
NSUP = NCHUNK // SK


SPLIT = 10


def _scatter_body(g_hbm, src_hbm, dst_hbm, zeros_hbm, out_hbm,
                  acc, sidx, didx, buf, sem0a, sem0b, sem1a, sem1b, *, half):
  sems = ((sem0a, sem0b), (sem1a, sem1b))
  c = lax.axis_index("c")
  s = lax.axis_index("s")
  rows = pl.ds(s * ROWS_PER_TILE, ROWS_PER_TILE)
  pltpu.sync_copy(zeros_hbm, acc.at[rows])
  plsc.subcore_barrier()

  table = g_hbm if half is None else g_hbm.at[half]
  nsup = jnp.where(c == 0, SPLIT, NSUP - SPLIT)
  sup0 = jnp.where(c == 0, 0, SPLIT)

  H = CHUNK // 2

  def gather(jl, slot, sem):
    sa, sb = sem
    pltpu.async_copy(table.at[sidx.at[jl, pl.ds(0, H)]],
                     buf.at[slot, pl.ds(0, H)], sa)
    pltpu.async_copy(table.at[sidx.at[jl, pl.ds(H, H)]],
                     buf.at[slot, pl.ds(H, H)], sb)

  def gwait(jl, slot, sem):
    sa, sb = sem
    pltpu.make_async_copy(table.at[sidx.at[jl, pl.ds(0, H)]],
                          buf.at[slot, pl.ds(0, H)], sa).wait()
    pltpu.make_async_copy(table.at[sidx.at[jl, pl.ds(H, H)]],
                          buf.at[slot, pl.ds(H, H)], sb).wait()

  def scat(jl, slot):
    pltpu.sync_copy(buf.at[slot], acc.at[didx.at[jl]], add=True)

  def sup_body(ul, _):
    u = sup0 + ul
    pltpu.sync_copy(src_hbm.at[s].at[pl.ds(u * SK, SK)], sidx)
    pltpu.sync_copy(dst_hbm.at[s].at[pl.ds(u * SK, SK)], didx)
    gather(0, 0, sems[0])

    def pair_body(i, _):
      j0 = 2 * i
      j1 = j0 + 1
      gather(j1, 1, sems[1])
      gwait(j0, 0, sems[0])
      scat(j0, 0)

      @pl.when(i < SK // 2 - 1)
      def _():
        gather(j0 + 2, 0, sems[0])
      gwait(j1, 1, sems[1])
      scat(j1, 1)
      return 0
    lax.fori_loop(0, SK // 2, pair_body, 0)
    return 0
  lax.fori_loop(0, nsup, sup_body, 0)

  plsc.subcore_barrier()
  pltpu.sync_copy(acc.at[rows], out_hbm.at[c].at[rows])


def _scatter(g, src3, dst3, half=None):
  zeros = jnp.zeros((ROWS_PER_TILE, DH), jnp.float32)
  body = functools.partial(_scatter_body, half=half)
  return pl.kernel(
      body,
      out_type=jax.ShapeDtypeStruct((NC, NPAD, DH), jnp.float32),
      mesh=_mesh,
      scratch_types=[
          pltpu.VMEM_SHARED((NPAD, DH), jnp.float32),
          pltpu.VMEM((SK, CHUNK), jnp.int32),
          pltpu.VMEM((SK, CHUNK), jnp.int32),
          pltpu.VMEM((2, CHUNK, DH), jnp.float32),
          pltpu.SemaphoreType.DMA,
          pltpu.SemaphoreType.DMA,
          pltpu.SemaphoreType.DMA,
          pltpu.SemaphoreType.DMA,
      ],
  )(g, src3, dst3, zeros)


TN = 2048
BATCH_PAD = 2 * N_GRAPHS


def _t1_body(deg_ref, x_ref, dinv_ref, g_ref):
  deg = deg_ref[0, :, 0] + deg_ref[1, :, 0] + 1.0
  dinv = lax.rsqrt(deg)[:, None]
  dinv_ref[...] = dinv
  g_ref[...] = x_ref[...] * dinv


def _t1(deg3, x):
  return pl.pallas_call(
      _t1_body,
      grid=(NPAD // TN,),
      in_specs=[
          pl.BlockSpec((NC, TN, DH), lambda i: (0, i, 0)),
          pl.BlockSpec((TN, D_IN), lambda i: (i, 0)),
      ],
      out_specs=[
          pl.BlockSpec((TN, 1), lambda i: (i, 0)),
          pl.BlockSpec((TN, D_IN), lambda i: (i, 0)),
      ],
      out_shape=[
          jax.ShapeDtypeStruct((NPAD, 1), jnp.float32),
          jax.ShapeDtypeStruct((NPAD, D_IN), jnp.float32),
      ],
  )(deg3, x)


def _dense1_body(s_ref, g_ref, dinv_ref, w_ref, b_ref, out_ref):
  dinv = dinv_ref[...]
  p = (s_ref[0] + s_ref[1] + g_ref[...]) * dinv
  h = jnp.dot(p, w_ref[...], preferred_element_type=jnp.float32) + b_ref[...]
  z = jnp.maximum(h, 0.0) * dinv
  out_ref[0] = z[:, :DH]
  out_ref[1] = z[:, DH:]


def _dense1(s, g, dinv, W, b):
  return pl.pallas_call(
      _dense1_body,
      grid=(NPAD // TN,),
      in_specs=[
          pl.BlockSpec((NC, TN, DH), lambda i: (0, i, 0)),
          pl.BlockSpec((TN, D_IN), lambda i: (i, 0)),
          pl.BlockSpec((TN, 1), lambda i: (i, 0)),
          pl.BlockSpec((D_IN, D_H), lambda i: (0, 0)),
          pl.BlockSpec((1, D_H), lambda i: (0, 0)),
      ],
      out_specs=pl.BlockSpec((NC, TN, DH), lambda i: (0, i, 0)),
      out_shape=jax.ShapeDtypeStruct((NC, NPAD, DH), jnp.float32),
  )(s, g, dinv, W, b)


def _dense2_body(sa_ref, sb_ref, g_ref, dinv_ref, w_ref, b_ref, out_ref):
  dinv = dinv_ref[...]
  p = (jnp.concatenate([sa_ref[0] + sa_ref[1] + g_ref[0],
                        sb_ref[0] + sb_ref[1] + g_ref[1]], axis=1)) * dinv
  h = jnp.dot(p, w_ref[...], preferred_element_type=jnp.float32) + b_ref[...]
  z = jnp.maximum(h, 0.0) * dinv
  out_ref[0] = z[:, :DH]
  out_ref[1] = z[:, DH:]


def _dense2(sa, sb, g_split, dinv, W, b):
  return pl.pallas_call(
      _dense2_body,
      grid=(NPAD // TN,),
      in_specs=[
          pl.BlockSpec((NC, TN, DH), lambda i: (0, i, 0)),
          pl.BlockSpec((NC, TN, DH), lambda i: (0, i, 0)),
          pl.BlockSpec((NC, TN, DH), lambda i: (0, i, 0)),
          pl.BlockSpec((TN, 1), lambda i: (i, 0)),
          pl.BlockSpec((D_H, D_H), lambda i: (0, 0)),
          pl.BlockSpec((1, D_H), lambda i: (0, 0)),
      ],
      out_specs=pl.BlockSpec((NC, TN, DH), lambda i: (0, i, 0)),
      out_shape=jax.ShapeDtypeStruct((NC, NPAD, DH), jnp.float32),
  )(sa, sb, g_split, dinv, W, b)


def _pool_body(sa_ref, sb_ref, g_ref, dinv_ref, batch_ref, pp_ref, cnt_ref):
  i = pl.program_id(0)
  dinv = dinv_ref[...]
  p = (jnp.concatenate([sa_ref[0] + sa_ref[1] + g_ref[0],
                        sb_ref[0] + sb_ref[1] + g_ref[1]], axis=1)) * dinv
  gids = lax.broadcasted_iota(jnp.int32, (1, N_GRAPHS), 1)
  mask = (batch_ref[...] == gids).astype(jnp.float32)

  @pl.when(i == 0)
  def _():
    pp_ref[...] = jnp.zeros_like(pp_ref)
    cnt_ref[...] = jnp.zeros_like(cnt_ref)

  pp_ref[...] += lax.dot_general(
      mask, p, (((0,), (0,)), ((), ())), preferred_element_type=jnp.float32)
  cnt_ref[...] += jnp.sum(mask, axis=0, keepdims=True)


def _pool(sa, sb, g_split, dinv, batch2):
  return pl.pallas_call(
      _pool_body,
      grid=(NPAD // TN,),
      in_specs=[
          pl.BlockSpec((NC, TN, DH), lambda i: (0, i, 0)),
          pl.BlockSpec((NC, TN, DH), lambda i: (0, i, 0)),
          pl.BlockSpec((NC, TN, DH), lambda i: (0, i, 0)),
          pl.BlockSpec((TN, 1), lambda i: (i, 0)),
          pl.BlockSpec((TN, 1), lambda i: (i, 0)),
      ],
      out_specs=[
          pl.BlockSpec((N_GRAPHS, D_H), lambda i: (0, 0)),
          pl.BlockSpec((1, N_GRAPHS), lambda i: (0, 0)),
      ],
      out_shape=[
          jax.ShapeDtypeStruct((N_GRAPHS, D_H), jnp.float32),
          jax.ShapeDtypeStruct((1, N_GRAPHS), jnp.float32),
      ],
  )(sa, sb, g_split, dinv, batch2)


def _head_body(pp_ref, cnt_ref, w3_ref, b3_ref, wl_ref, bl_ref, out_ref):
  cnt = cnt_ref[...][0][:, None]
  pooled = (jnp.dot(pp_ref[...], w3_ref[...], preferred_element_type=jnp.float32)
            + cnt * b3_ref[...])
  out_ref[...] = (jnp.dot(pooled, wl_ref[...], preferred_element_type=jnp.float32)
                  + bl_ref[...])


def _head(pp, cnt, W3, b3, Wl, bl):
  return pl.pallas_call(
      _head_body,
      out_shape=jax.ShapeDtypeStruct((N_GRAPHS, N_CLASSES), jnp.float32),
  )(pp, cnt, W3, b3, Wl, bl)


def kernel(x, edge_index, batch, W1, b1, W2, b2, W3, b3, Wl, bl):
  src = edge_index[0]
  dst = edge_index[1]
  srcp = jnp.concatenate([src, jnp.zeros((EPAD - E,), jnp.int32)])
  dstp = jnp.concatenate([dst, jnp.full((EPAD - E,), DUMMY, jnp.int32)])
  src3 = srcp.reshape(NS, NCHUNK, CHUNK)
  dst3 = dstp.reshape(NS, NCHUNK, CHUNK)

  xpad = jnp.concatenate([x, jnp.zeros((NPAD - N, D_IN), jnp.float32)])
  batchpad = jnp.concatenate(
      [batch, jnp.full((NPAD - N,), BATCH_PAD, jnp.int32)]).reshape(NPAD, 1)

  deg3 = _deg(dst3)
  dinv, g0 = _t1(deg3, xpad)
  s0 = _scatter(g0, src3, dst3)
  z1 = _dense1(s0, g0, dinv, W1, b1.reshape(1, D_H))
  s1a = _scatter(z1, src3, dst3, half=0)
  s1b = _scatter(z1, src3, dst3, half=1)
  z2 = _dense2(s1a, s1b, z1, dinv, W2, b2.reshape(1, D_H))
  s2a = _scatter(z2, src3, dst3, half=0)
  s2b = _scatter(z2, src3, dst3, half=1)
  pp, cnt = _pool(s2a, s2b, z2, dinv, batchpad)
  return _head(pp, cnt, W3, b3.reshape(1, D_H), Wl, bl.reshape(1, N_CLASSES))


# speedup vs baseline: 1.5375x; 1.5375x over previous
import functools

import jax
import jax.numpy as jnp
from jax import lax
from jax.experimental import pallas as pl
from jax.experimental.pallas import tpu as pltpu
from jax.experimental.pallas import tpu_sc as plsc

N = 10000
E = 320000
D_IN = 128
D_H = 256
N_CLASSES = 10
N_GRAPHS = 128

NC, NS = 2, 16
CHUNK = 128
NCHUNK = 160
EPT = NCHUNK * CHUNK
EPAD = NS * EPT
NPAD = 10240
DUMMY = N
ROWS_PER_TILE = NPAD // NS
DH = 128

_mesh = plsc.VectorSubcoreMesh(
    core_axis_name="c", subcore_axis_name="s", num_cores=NC, num_subcores=NS)


def _deg_body(dst_hbm, ones_hbm, zeros_hbm, out_hbm, acc, dstv, onesv):
  c = lax.axis_index("c")
  s = lax.axis_index("s")
  pltpu.sync_copy(dst_hbm.at[s], dstv)
  pltpu.sync_copy(ones_hbm, onesv)
  rows = pl.ds(s * ROWS_PER_TILE, ROWS_PER_TILE)
  pltpu.sync_copy(zeros_hbm, acc.at[rows])
  plsc.subcore_barrier()

  base = c * (NCHUNK // 2)

  def hist_body(j, _):
    pltpu.sync_copy(onesv, acc.at[dstv.at[base + j]], add=True)
    return 0
  lax.fori_loop(0, NCHUNK // 2, hist_body, 0)

  plsc.subcore_barrier()
  pltpu.sync_copy(acc.at[rows], out_hbm.at[c].at[rows])


def _deg(dst3):
  ones = jnp.ones((CHUNK, DH), jnp.float32)
  zeros = jnp.zeros((ROWS_PER_TILE, DH), jnp.float32)
  return pl.kernel(
      _deg_body,
      out_type=jax.ShapeDtypeStruct((NC, NPAD, DH), jnp.float32),
      mesh=_mesh,
      scratch_types=[
          pltpu.VMEM_SHARED((NPAD, DH), jnp.float32),
          pltpu.VMEM((NCHUNK, CHUNK), jnp.int32),
          pltpu.VMEM((CHUNK, DH), jnp.float32),
      ],
  )(dst3, ones, zeros)


SK = 16
NSUP = NCHUNK // SK


def _scatter_body(g_hbm, src_hbm, dst_hbm, zeros_hbm, out_hbm,
                  acc, sidx, didx, buf, sem0, sem1, *, edge_split):
  c = lax.axis_index("c")
  s = lax.axis_index("s")
  rows = pl.ds(s * ROWS_PER_TILE, ROWS_PER_TILE)
  pltpu.sync_copy(zeros_hbm, acc.at[rows])
  plsc.subcore_barrier()

  if edge_split:
    table = g_hbm
    nsup = jnp.where(c == 0, 8, NSUP - 8)
    sup0 = jnp.where(c == 0, 0, 8)
  else:
    table = g_hbm.at[c]
    nsup = NSUP
    sup0 = 0

  def gather(jl, slot, sem):
    pltpu.async_copy(table.at[sidx.at[jl]], buf.at[slot], sem)

  def gwait(jl, slot, sem):
    pltpu.make_async_copy(table.at[sidx.at[jl]], buf.at[slot], sem).wait()

  def scat(jl, slot):
    pltpu.sync_copy(buf.at[slot], acc.at[didx.at[jl]], add=True)

  def sup_body(ul, _):
    u = sup0 + ul
    pltpu.sync_copy(src_hbm.at[s].at[pl.ds(u * SK, SK)], sidx)
    pltpu.sync_copy(dst_hbm.at[s].at[pl.ds(u * SK, SK)], didx)
    gather(0, 0, sem0)

    def pair_body(i, _):
      j0 = 2 * i
      j1 = j0 + 1
      gather(j1, 1, sem1)
      gwait(j0, 0, sem0)
      scat(j0, 0)

      @pl.when(i < SK // 2 - 1)
      def _():
        gather(j0 + 2, 0, sem0)
      gwait(j1, 1, sem1)
      scat(j1, 1)
      return 0
    lax.fori_loop(0, SK // 2, pair_body, 0)
    return 0
  lax.fori_loop(0, nsup, sup_body, 0)

  plsc.subcore_barrier()
  pltpu.sync_copy(acc.at[rows], out_hbm.at[c].at[rows])


def _scatter(g, src3, dst3, edge_split):
  zeros = jnp.zeros((ROWS_PER_TILE, DH), jnp.float32)
  body = functools.partial(_scatter_body, edge_split=edge_split)
  return pl.kernel(
      body,
      out_type=jax.ShapeDtypeStruct((NC, NPAD, DH), jnp.float32),
      mesh=_mesh,
      scratch_types=[
          pltpu.VMEM_SHARED((NPAD, DH), jnp.float32),
          pltpu.VMEM((SK, CHUNK), jnp.int32),
          pltpu.VMEM((SK, CHUNK), jnp.int32),
          pltpu.VMEM((2, CHUNK, DH), jnp.float32),
          pltpu.SemaphoreType.DMA,
          pltpu.SemaphoreType.DMA,
      ],
  )(g, src3, dst3, zeros)


TN = 2048
BATCH_PAD = 2 * N_GRAPHS


def _t1_body(deg_ref, x_ref, dinv_ref, g_ref):
  deg = deg_ref[0, :, 0] + deg_ref[1, :, 0] + 1.0
  dinv = lax.rsqrt(deg)[:, None]
  dinv_ref[...] = dinv
  g_ref[...] = x_ref[...] * dinv


def _t1(deg3, x):
  return pl.pallas_call(
      _t1_body,
      grid=(NPAD // TN,),
      in_specs=[
          pl.BlockSpec((NC, TN, DH), lambda i: (0, i, 0)),
          pl.BlockSpec((TN, D_IN), lambda i: (i, 0)),
      ],
      out_specs=[
          pl.BlockSpec((TN, 1), lambda i: (i, 0)),
          pl.BlockSpec((TN, D_IN), lambda i: (i, 0)),
      ],
      out_shape=[
          jax.ShapeDtypeStruct((NPAD, 1), jnp.float32),
          jax.ShapeDtypeStruct((NPAD, D_IN), jnp.float32),
      ],
  )(deg3, x)


def _dense1_body(s_ref, g_ref, dinv_ref, w_ref, b_ref, out_ref):
  dinv = dinv_ref[...]
  p = (s_ref[0] + s_ref[1] + g_ref[...]) * dinv
  h = jnp.dot(p, w_ref[...], preferred_element_type=jnp.float32) + b_ref[...]
  z = jnp.maximum(h, 0.0) * dinv
  out_ref[0] = z[:, :DH]
  out_ref[1] = z[:, DH:]


def _dense1(s, g, dinv, W, b):
  return pl.pallas_call(
      _dense1_body,
      grid=(NPAD // TN,),
      in_specs=[
          pl.BlockSpec((NC, TN, DH), lambda i: (0, i, 0)),
          pl.BlockSpec((TN, D_IN), lambda i: (i, 0)),
          pl.BlockSpec((TN, 1), lambda i: (i, 0)),
          pl.BlockSpec((D_IN, D_H), lambda i: (0, 0)),
          pl.BlockSpec((1, D_H), lambda i: (0, 0)),
      ],
      out_specs=pl.BlockSpec((NC, TN, DH), lambda i: (0, i, 0)),
      out_shape=jax.ShapeDtypeStruct((NC, NPAD, DH), jnp.float32),
  )(s, g, dinv, W, b)


def _dense2_body(s_ref, g_ref, dinv_ref, w_ref, b_ref, out_ref):
  dinv = dinv_ref[...]
  p = (jnp.concatenate([s_ref[0], s_ref[1]], axis=1)
       + jnp.concatenate([g_ref[0], g_ref[1]], axis=1)) * dinv
  h = jnp.dot(p, w_ref[...], preferred_element_type=jnp.float32) + b_ref[...]
  z = jnp.maximum(h, 0.0) * dinv
  out_ref[0] = z[:, :DH]
  out_ref[1] = z[:, DH:]


def _dense2(s_split, g_split, dinv, W, b):
  return pl.pallas_call(
      _dense2_body,
      grid=(NPAD // TN,),
      in_specs=[
          pl.BlockSpec((NC, TN, DH), lambda i: (0, i, 0)),
          pl.BlockSpec((NC, TN, DH), lambda i: (0, i, 0)),
          pl.BlockSpec((TN, 1), lambda i: (i, 0)),
          pl.BlockSpec((D_H, D_H), lambda i: (0, 0)),
          pl.BlockSpec((1, D_H), lambda i: (0, 0)),
      ],
      out_specs=pl.BlockSpec((NC, TN, DH), lambda i: (0, i, 0)),
      out_shape=jax.ShapeDtypeStruct((NC, NPAD, DH), jnp.float32),
  )(s_split, g_split, dinv, W, b)


def _pool_body(s_ref, g_ref, dinv_ref, batch_ref, pp_ref, cnt_ref):
  i = pl.program_id(0)
  dinv = dinv_ref[...]
  p = (jnp.concatenate([s_ref[0], s_ref[1]], axis=1)
       + jnp.concatenate([g_ref[0], g_ref[1]], axis=1)) * dinv
  gids = lax.broadcasted_iota(jnp.int32, (1, N_GRAPHS), 1)
  mask = (batch_ref[...] == gids).astype(jnp.float32)

  @pl.when(i == 0)
  def _():
    pp_ref[...] = jnp.zeros_like(pp_ref)
    cnt_ref[...] = jnp.zeros_like(cnt_ref)

  pp_ref[...] += lax.dot_general(
      mask, p, (((0,), (0,)), ((), ())), preferred_element_type=jnp.float32)
  cnt_ref[...] += jnp.sum(mask, axis=0, keepdims=True)


def _pool(s_split, g_split, dinv, batch2):
  return pl.pallas_call(
      _pool_body,
      grid=(NPAD // TN,),
      in_specs=[
          pl.BlockSpec((NC, TN, DH), lambda i: (0, i, 0)),
          pl.BlockSpec((NC, TN, DH), lambda i: (0, i, 0)),
          pl.BlockSpec((TN, 1), lambda i: (i, 0)),
          pl.BlockSpec((TN, 1), lambda i: (i, 0)),
      ],
      out_specs=[
          pl.BlockSpec((N_GRAPHS, D_H), lambda i: (0, 0)),
          pl.BlockSpec((1, N_GRAPHS), lambda i: (0, 0)),
      ],
      out_shape=[
          jax.ShapeDtypeStruct((N_GRAPHS, D_H), jnp.float32),
          jax.ShapeDtypeStruct((1, N_GRAPHS), jnp.float32),
      ],
  )(s_split, g_split, dinv, batch2)


def _head_body(pp_ref, cnt_ref, w3_ref, b3_ref, wl_ref, bl_ref, out_ref):
  cnt = cnt_ref[...][0][:, None]
  pooled = (jnp.dot(pp_ref[...], w3_ref[...], preferred_element_type=jnp.float32)
            + cnt * b3_ref[...])
  out_ref[...] = (jnp.dot(pooled, wl_ref[...], preferred_element_type=jnp.float32)
                  + bl_ref[...])


def _head(pp, cnt, W3, b3, Wl, bl):
  return pl.pallas_call(
      _head_body,
      out_shape=jax.ShapeDtypeStruct((N_GRAPHS, N_CLASSES), jnp.float32),
  )(pp, cnt, W3, b3, Wl, bl)


def kernel(x, edge_index, batch, W1, b1, W2, b2, W3, b3, Wl, bl):
  src = edge_index[0]
  dst = edge_index[1]
  srcp = jnp.concatenate([src, jnp.zeros((EPAD - E,), jnp.int32)])
  dstp = jnp.concatenate([dst, jnp.full((EPAD - E,), DUMMY, jnp.int32)])
  src3 = srcp.reshape(NS, NCHUNK, CHUNK)
  dst3 = dstp.reshape(NS, NCHUNK, CHUNK)

  xpad = jnp.concatenate([x, jnp.zeros((NPAD - N, D_IN), jnp.float32)])
  batchpad = jnp.concatenate(
      [batch, jnp.full((NPAD - N,), BATCH_PAD, jnp.int32)]).reshape(NPAD, 1)

  deg3 = _deg(dst3)
  dinv, g0 = _t1(deg3, xpad)
  s0 = _scatter(g0, src3, dst3, edge_split=True)
  z1 = _dense1(s0, g0, dinv, W1, b1.reshape(1, D_H))
  s1 = _scatter(z1, src3, dst3, edge_split=False)
  z2 = _dense2(s1, z1, dinv, W2, b2.reshape(1, D_H))
  s2 = _scatter(z2, src3, dst3, edge_split=False)
  pp, cnt = _pool(s2, z2, dinv, batchpad)
  return _head(pp, cnt, W3, b3.reshape(1, D_H), Wl, bl.reshape(1, N_CLASSES))
